# Initial kernel scaffold; baseline (speedup 1.0000x reference)
import jax
import jax.numpy as jnp
from jax.experimental import pallas as pl


def kernel(x, temperature, codebook):
    raise NotImplementedError("write your pallas kernel here")


# speedup vs baseline: 1.0387x; 1.0387x over previous
import functools

import jax
import jax.numpy as jnp
from jax import lax
from jax.experimental import pallas as pl
from jax.experimental.pallas import tpu as pltpu
from jax.experimental.pallas import tpu_sc as plsc

B_TOK = 8192
D = 256
K = 8192
COMMIT_W = 0.25

BX = 2048
BC = 1024
NXB = B_TOK // BX
NCB = K // BC

SC_CORES = 2
SC_SUBCORES = 16
NW = SC_CORES * SC_SUBCORES
BPW = B_TOK // NW
GCHUNK = 128


def _dist_argmin_body(x_ref, cb_ref, ids_ref, loss_ref, minval, minidx, acc):
    i = pl.program_id(0)
    j = pl.program_id(1)
    x = x_ref[...]
    c = cb_ref[...]
    xc = lax.dot_general(
        x, c, (((1,), (1,)), ((), ())),
        preferred_element_type=jnp.float32,
        precision=lax.Precision.DEFAULT,
    )
    xn = jnp.sum(x * x, axis=1, keepdims=True)
    cn = jnp.sum(c * c, axis=1, keepdims=True)
    dist = (xn + cn.T) - 2.0 * xc
    bmin = jnp.min(dist, axis=1, keepdims=True)
    iota = lax.broadcasted_iota(jnp.int32, dist.shape, 1)
    bidx = jnp.min(jnp.where(dist == bmin, iota, K), axis=1, keepdims=True)
    bidx = bidx + j * BC

    @pl.when(j == 0)
    def _():
        minval[...] = bmin
        minidx[...] = bidx

    @pl.when(j > 0)
    def _():
        better = bmin < minval[...]
        minval[...] = jnp.where(better, bmin, minval[...])
        minidx[...] = jnp.where(better, bidx, minidx[...])

    @pl.when(j == NCB - 1)
    def _():
        ids_ref[...] = minidx[...]
        part = jnp.sum(minval[...])
        tot = jnp.where(i == 0, part, acc[0, 0] + part)
        acc[0, 0] = tot

        @pl.when(i == NXB - 1)
        def _():
            loss_ref[0, 0] = tot * ((1.0 + COMMIT_W) / B_TOK)


def _dist_argmin(x, codebook):
    return pl.pallas_call(
        _dist_argmin_body,
        grid=(NXB, NCB),
        in_specs=[
            pl.BlockSpec((BX, D), lambda i, j: (i, 0)),
            pl.BlockSpec((BC, D), lambda i, j: (j, 0)),
        ],
        out_specs=[
            pl.BlockSpec((BX, 1), lambda i, j: (i, 0)),
            pl.BlockSpec(memory_space=pltpu.SMEM),
        ],
        out_shape=[
            jax.ShapeDtypeStruct((B_TOK, 1), jnp.int32),
            jax.ShapeDtypeStruct((1, 1), jnp.float32),
        ],
        scratch_shapes=[
            pltpu.VMEM((BX, 1), jnp.float32),
            pltpu.VMEM((BX, 1), jnp.int32),
            pltpu.SMEM((1, 1), jnp.float32),
        ],
    )(x, codebook)


def _gather_body(cb_hbm, ids_hbm, out_hbm, idx_v, rows_v, sem):
    wid = lax.axis_index("s") * SC_CORES + lax.axis_index("c")
    base = wid * BPW
    for t in range(BPW // GCHUNK):
        off = base + t * GCHUNK
        pltpu.sync_copy(ids_hbm.at[pl.ds(off, GCHUNK)], idx_v)
        pltpu.async_copy(cb_hbm.at[idx_v], rows_v, sem).wait()
        pltpu.sync_copy(rows_v, out_hbm.at[pl.ds(off, GCHUNK)])


def _sc_gather(codebook, ids):
    mesh = plsc.VectorSubcoreMesh(
        core_axis_name="c", subcore_axis_name="s",
        num_cores=SC_CORES, num_subcores=SC_SUBCORES,
    )
    return pl.kernel(
        _gather_body,
        out_type=jax.ShapeDtypeStruct((B_TOK, D), jnp.float32),
        mesh=mesh,
        scratch_types=[
            pltpu.VMEM((GCHUNK,), jnp.int32),
            pltpu.VMEM((GCHUNK, D), jnp.float32),
            pltpu.SemaphoreType.DMA,
        ],
    )(codebook, ids)


def kernel(x, temperature, codebook):
    ids2d, loss = _dist_argmin(x, codebook)
    ids = ids2d.reshape(B_TOK)
    emb_out = _sc_gather(codebook, ids)
    return emb_out, ids, loss.reshape(())


# speedup vs baseline: 1.7060x; 1.6424x over previous
import functools

import jax
import jax.numpy as jnp
from jax import lax
from jax.experimental import pallas as pl
from jax.experimental.pallas import tpu as pltpu
from jax.experimental.pallas import tpu_sc as plsc

B_TOK = 8192
D = 256
K = 8192
COMMIT_W = 0.25

BX = 2048
BC = 1024
NXB = B_TOK // BX
NCB = K // BC

SC_CORES = 2
SC_SUBCORES = 16
NW = SC_CORES * SC_SUBCORES
BPW = B_TOK // NW
GCHUNK = 128


def _dist_argmin_body(x_ref, cb_ref, ids_ref, loss_ref, x2, minval, minidx,
                      acc, xsum):
    i = pl.program_id(0)
    j = pl.program_id(1)

    @pl.when(j == 0)
    def _():
        xx = x_ref[...]
        x2[...] = xx * -2.0
        xsum[0, 0] = jnp.sum(xx * xx)

    c = cb_ref[...]
    cn = jnp.sum(c * c, axis=1, keepdims=True)
    xc2 = lax.dot_general(
        c, x2[...], (((1,), (1,)), ((), ())),
        preferred_element_type=jnp.float32,
        precision=lax.Precision.DEFAULT,
    )
    distT = xc2 + cn
    bmin = jnp.min(distT, axis=0)
    bidx = jnp.argmin(distT, axis=0).astype(jnp.int32) + j * BC

    @pl.when(j == 0)
    def _():
        minval[...] = bmin
        minidx[...] = bidx

    @pl.when(j > 0)
    def _():
        better = bmin < minval[...]
        minval[...] = jnp.where(better, bmin, minval[...])
        minidx[...] = jnp.where(better, bidx, minidx[...])

    @pl.when(j == NCB - 1)
    def _():
        ids_ref[...] = minidx[...]
        part = jnp.sum(minval[...]) + xsum[0, 0]
        tot = jnp.where(i == 0, part, acc[0, 0] + part)
        acc[0, 0] = tot

        @pl.when(i == NXB - 1)
        def _():
            loss_ref[0, 0] = tot * ((1.0 + COMMIT_W) / B_TOK)


def _dist_argmin(x, codebook):
    return pl.pallas_call(
        _dist_argmin_body,
        grid=(NXB, NCB),
        in_specs=[
            pl.BlockSpec((BX, D), lambda i, j: (i, 0)),
            pl.BlockSpec((BC, D), lambda i, j: (j, 0)),
        ],
        out_specs=[
            pl.BlockSpec((BX,), lambda i, j: (i,)),
            pl.BlockSpec(memory_space=pltpu.SMEM),
        ],
        out_shape=[
            jax.ShapeDtypeStruct((B_TOK,), jnp.int32),
            jax.ShapeDtypeStruct((1, 1), jnp.float32),
        ],
        scratch_shapes=[
            pltpu.VMEM((BX, D), jnp.float32),
            pltpu.VMEM((BX,), jnp.float32),
            pltpu.VMEM((BX,), jnp.int32),
            pltpu.SMEM((1, 1), jnp.float32),
            pltpu.SMEM((1, 1), jnp.float32),
        ],
    )(x, codebook)


def _gather_body(cb_hbm, ids_hbm, out_hbm, idx_v, rows_v, sem):
    wid = lax.axis_index("s") * SC_CORES + lax.axis_index("c")
    base = wid * BPW
    for t in range(BPW // GCHUNK):
        off = base + t * GCHUNK
        pltpu.sync_copy(ids_hbm.at[pl.ds(off, GCHUNK)], idx_v)
        pltpu.async_copy(cb_hbm.at[idx_v], rows_v, sem).wait()
        pltpu.sync_copy(rows_v, out_hbm.at[pl.ds(off, GCHUNK)])


def _sc_gather(codebook, ids):
    mesh = plsc.VectorSubcoreMesh(
        core_axis_name="c", subcore_axis_name="s",
        num_cores=SC_CORES, num_subcores=SC_SUBCORES,
    )
    return pl.kernel(
        _gather_body,
        out_type=jax.ShapeDtypeStruct((B_TOK, D), jnp.float32),
        mesh=mesh,
        scratch_types=[
            pltpu.VMEM((GCHUNK,), jnp.int32),
            pltpu.VMEM((GCHUNK, D), jnp.float32),
            pltpu.SemaphoreType.DMA,
        ],
    )(codebook, ids)


def kernel(x, temperature, codebook):
    ids, loss = _dist_argmin(x, codebook)
    emb_out = _sc_gather(codebook, ids)
    return emb_out, ids, loss.reshape(())


# speedup vs baseline: 1.8350x; 1.0756x over previous
import functools

import jax
import jax.numpy as jnp
from jax import lax
from jax.experimental import pallas as pl
from jax.experimental.pallas import tpu as pltpu
from jax.experimental.pallas import tpu_sc as plsc

B_TOK = 8192
D = 256
K = 8192
COMMIT_W = 0.25

BX = 2048
BC = 2048
NXB = B_TOK // BX
NCB = K // BC

SC_CORES = 2
SC_SUBCORES = 16
NW = SC_CORES * SC_SUBCORES
BPW = B_TOK // NW
GCHUNK = 128


def _dist_argmin_body(x_ref, cb_ref, ids_ref, loss_ref, x2, minval, minidx,
                      acc, xsum):
    i = pl.program_id(0)
    j = pl.program_id(1)

    @pl.when(j == 0)
    def _():
        xx = x_ref[...]
        x2[...] = xx * -2.0
        xsum[0, 0] = jnp.sum(xx * xx)

    c = cb_ref[...]
    cn = jnp.sum(c * c, axis=1, keepdims=True)
    xc2 = lax.dot_general(
        c, x2[...], (((1,), (1,)), ((), ())),
        preferred_element_type=jnp.float32,
        precision=lax.Precision.DEFAULT,
    )
    distT = xc2 + cn
    bmin = jnp.min(distT, axis=0)
    bidx = jnp.argmin(distT, axis=0).astype(jnp.int32) + j * BC

    @pl.when(j == 0)
    def _():
        minval[...] = bmin
        minidx[...] = bidx

    @pl.when(j > 0)
    def _():
        better = bmin < minval[...]
        minval[...] = jnp.where(better, bmin, minval[...])
        minidx[...] = jnp.where(better, bidx, minidx[...])

    @pl.when(j == NCB - 1)
    def _():
        ids_ref[...] = minidx[...]
        part = jnp.sum(minval[...]) + xsum[0, 0]
        tot = jnp.where(i == 0, part, acc[0, 0] + part)
        acc[0, 0] = tot

        @pl.when(i == NXB - 1)
        def _():
            loss_ref[0, 0] = tot * ((1.0 + COMMIT_W) / B_TOK)


def _dist_argmin(x, codebook):
    return pl.pallas_call(
        _dist_argmin_body,
        grid=(NXB, NCB),
        in_specs=[
            pl.BlockSpec((BX, D), lambda i, j: (i, 0)),
            pl.BlockSpec((BC, D), lambda i, j: (j, 0)),
        ],
        out_specs=[
            pl.BlockSpec((BX,), lambda i, j: (i,)),
            pl.BlockSpec(memory_space=pltpu.SMEM),
        ],
        out_shape=[
            jax.ShapeDtypeStruct((B_TOK,), jnp.int32),
            jax.ShapeDtypeStruct((1, 1), jnp.float32),
        ],
        scratch_shapes=[
            pltpu.VMEM((BX, D), jnp.float32),
            pltpu.VMEM((BX,), jnp.float32),
            pltpu.VMEM((BX,), jnp.int32),
            pltpu.SMEM((1, 1), jnp.float32),
            pltpu.SMEM((1, 1), jnp.float32),
        ],
    )(x, codebook)


def _gather_body(cb_hbm, ids_hbm, out_hbm, idx_v, rows_v, sem):
    wid = lax.axis_index("s") * SC_CORES + lax.axis_index("c")
    base = wid * BPW
    for t in range(BPW // GCHUNK):
        off = base + t * GCHUNK
        pltpu.sync_copy(ids_hbm.at[pl.ds(off, GCHUNK)], idx_v)
        pltpu.async_copy(cb_hbm.at[idx_v], rows_v, sem).wait()
        pltpu.sync_copy(rows_v, out_hbm.at[pl.ds(off, GCHUNK)])


def _sc_gather(codebook, ids):
    mesh = plsc.VectorSubcoreMesh(
        core_axis_name="c", subcore_axis_name="s",
        num_cores=SC_CORES, num_subcores=SC_SUBCORES,
    )
    return pl.kernel(
        _gather_body,
        out_type=jax.ShapeDtypeStruct((B_TOK, D), jnp.float32),
        mesh=mesh,
        scratch_types=[
            pltpu.VMEM((GCHUNK,), jnp.int32),
            pltpu.VMEM((GCHUNK, D), jnp.float32),
            pltpu.SemaphoreType.DMA,
        ],
    )(codebook, ids)


def kernel(x, temperature, codebook):
    ids, loss = _dist_argmin(x, codebook)
    emb_out = _sc_gather(codebook, ids)
    return emb_out, ids, loss.reshape(())


# speedup vs baseline: 1.8935x; 1.0319x over previous
import functools

import jax
import jax.numpy as jnp
from jax import lax
from jax.experimental import pallas as pl
from jax.experimental.pallas import tpu as pltpu
from jax.experimental.pallas import tpu_sc as plsc

B_TOK = 8192
D = 256
K = 8192
COMMIT_W = 0.25

BX = 2048
BC = 4096
NXB = B_TOK // BX
NCB = K // BC

SC_CORES = 2
SC_SUBCORES = 16
NW = SC_CORES * SC_SUBCORES
BPW = B_TOK // NW
GCHUNK = 128


def _dist_argmin_body(x_ref, cb_ref, ids_ref, loss_ref, x2, minval, minidx,
                      acc, xsum):
    i = pl.program_id(0)
    j = pl.program_id(1)

    @pl.when(j == 0)
    def _():
        xx = x_ref[...]
        x2[...] = xx * -2.0
        xsum[0, 0] = jnp.sum(xx * xx)

    c = cb_ref[...]
    cn = jnp.sum(c * c, axis=1, keepdims=True)
    xc2 = lax.dot_general(
        c, x2[...], (((1,), (1,)), ((), ())),
        preferred_element_type=jnp.float32,
        precision=lax.Precision.DEFAULT,
    )
    distT = xc2 + cn
    bmin = jnp.min(distT, axis=0)
    bidx = jnp.argmin(distT, axis=0).astype(jnp.int32) + j * BC

    @pl.when(j == 0)
    def _():
        minval[...] = bmin
        minidx[...] = bidx

    @pl.when(j > 0)
    def _():
        better = bmin < minval[...]
        minval[...] = jnp.where(better, bmin, minval[...])
        minidx[...] = jnp.where(better, bidx, minidx[...])

    @pl.when(j == NCB - 1)
    def _():
        ids_ref[...] = minidx[...]
        part = jnp.sum(minval[...]) + xsum[0, 0]
        tot = jnp.where(i == 0, part, acc[0, 0] + part)
        acc[0, 0] = tot

        @pl.when(i == NXB - 1)
        def _():
            loss_ref[0, 0] = tot * ((1.0 + COMMIT_W) / B_TOK)


def _dist_argmin(x, codebook):
    return pl.pallas_call(
        _dist_argmin_body,
        grid=(NXB, NCB),
        in_specs=[
            pl.BlockSpec((BX, D), lambda i, j: (i, 0)),
            pl.BlockSpec((BC, D), lambda i, j: (j, 0)),
        ],
        out_specs=[
            pl.BlockSpec((BX,), lambda i, j: (i,)),
            pl.BlockSpec(memory_space=pltpu.SMEM),
        ],
        out_shape=[
            jax.ShapeDtypeStruct((B_TOK,), jnp.int32),
            jax.ShapeDtypeStruct((1, 1), jnp.float32),
        ],
        scratch_shapes=[
            pltpu.VMEM((BX, D), jnp.float32),
            pltpu.VMEM((BX,), jnp.float32),
            pltpu.VMEM((BX,), jnp.int32),
            pltpu.SMEM((1, 1), jnp.float32),
            pltpu.SMEM((1, 1), jnp.float32),
        ],
    )(x, codebook)


def _gather_body(cb_hbm, ids_hbm, out_hbm, idx_v, rows_v, sem):
    wid = lax.axis_index("s") * SC_CORES + lax.axis_index("c")
    base = wid * BPW
    for t in range(BPW // GCHUNK):
        off = base + t * GCHUNK
        pltpu.sync_copy(ids_hbm.at[pl.ds(off, GCHUNK)], idx_v)
        pltpu.async_copy(cb_hbm.at[idx_v], rows_v, sem).wait()
        pltpu.sync_copy(rows_v, out_hbm.at[pl.ds(off, GCHUNK)])


def _sc_gather(codebook, ids):
    mesh = plsc.VectorSubcoreMesh(
        core_axis_name="c", subcore_axis_name="s",
        num_cores=SC_CORES, num_subcores=SC_SUBCORES,
    )
    return pl.kernel(
        _gather_body,
        out_type=jax.ShapeDtypeStruct((B_TOK, D), jnp.float32),
        mesh=mesh,
        scratch_types=[
            pltpu.VMEM((GCHUNK,), jnp.int32),
            pltpu.VMEM((GCHUNK, D), jnp.float32),
            pltpu.SemaphoreType.DMA,
        ],
    )(codebook, ids)


def kernel(x, temperature, codebook):
    ids, loss = _dist_argmin(x, codebook)
    emb_out = _sc_gather(codebook, ids)
    return emb_out, ids, loss.reshape(())
